# Initial kernel scaffold; baseline (speedup 1.0000x reference)
import jax
import jax.numpy as jnp
from jax.experimental import pallas as pl


def kernel(states, adj_list, adj_list_mask, lut, W, b):
    raise NotImplementedError("write your pallas kernel here")


# speedup vs baseline: 11.0682x; 11.0682x over previous
import functools

import jax
import jax.numpy as jnp
from jax import lax
from jax.experimental import pallas as pl
from jax.experimental.pallas import tpu as pltpu
from jax.experimental.pallas import tpu_sc as plsc

N_NODES = 100000
N_INPUT = 1024
K_MAX = 8
N_OUT = 128
BATCH = 32

NPAD = 100352
NRES = NPAD - N_INPUT
BLK = 1024
GRID = NRES // BLK
E = K_MAX * NRES
NW = 32
PER_W = E // NW
CHUNK = PER_W // 4
VECS = CHUNK // 16


def _pack_body(s_ref, o_ref):
    blk = s_ref[...]
    sh = lax.broadcasted_iota(jnp.int32, (BATCH, 1), 0)
    w = jnp.sum(jnp.left_shift(blk, sh), axis=0)
    o_ref[...] = w.reshape(1, 16, 128)


def _gather_body(table_hbm, idx_hbm, out_hbm, table_v, idx_v, out_v):
    wid = lax.axis_index("s") * 2 + lax.axis_index("c")
    pltpu.sync_copy(table_hbm, table_v)
    for c in range(4):
        base = wid * PER_W + c * CHUNK
        pltpu.sync_copy(idx_hbm.at[pl.ds(base, CHUNK)], idx_v)

        def body(i, carry):
            iv = idx_v[pl.ds(i * 16, 16)]
            out_v[pl.ds(i * 16, 16)] = plsc.load_gather(table_v, [iv])
            return carry

        lax.fori_loop(0, VECS, body, 0)
        pltpu.sync_copy(out_v, out_hbm.at[pl.ds(base, CHUNK)])


def _main_body(lut_ref, g_ref, m_ref, p_ref, w_ref, bias_ref, out_ref):
    i = pl.program_id(0)
    lutblk = lut_ref[...]
    cexp = lax.broadcasted_iota(jnp.int32, (1, 256), 1) % 16
    s = jnp.left_shift(lutblk, cexp).astype(jnp.float32)
    jidx = lax.broadcasted_iota(jnp.int32, (16, 256), 0)
    cidx = lax.broadcasted_iota(jnp.int32, (16, 256), 1)
    m16 = (cidx // 16 == jidx).astype(jnp.float32)
    h = lax.dot_general(m16, s, (((1,), (1,)), ((), ())),
                        preferred_element_type=jnp.float32)
    h = h.astype(jnp.int32).reshape(16, 8, 128)
    words = [h[2 * j] + jnp.left_shift(h[2 * j + 1], 16) for j in range(8)]

    g = g_ref[...].reshape(K_MAX, 8, 128)
    m = m_ref[...].reshape(K_MAX, 8, 128)
    gm = [g[k] * m[k] for k in range(K_MAX)]
    msum = m[0]
    for k in range(1, K_MAX):
        msum = msum + m[k]
    nn = msum == 0
    pk = p_ref[...].reshape(8, 128)

    rows = []
    for bb in range(BATCH):
        acc = None
        for k in range(K_MAX):
            bit = jnp.right_shift(gm[k], bb) & 1
            acc = bit if acc is None else acc + acc + bit
        idx = acc
        wsel = jnp.right_shift(idx, 5)
        sel = words[0]
        for j in range(1, 8):
            sel = jnp.where(wsel == j, words[j], sel)
        lbit = jnp.right_shift(sel, idx & 31) & 1
        old = jnp.right_shift(pk, bb) & 1
        rows.append(jnp.where(nn, old, lbit))
    res = jnp.stack(rows).reshape(BATCH, BLK).astype(jnp.float32)
    part = jnp.dot(res, w_ref[...], preferred_element_type=jnp.float32)

    @pl.when(i == 0)
    def _():
        out_ref[...] = part

    @pl.when(i > 0)
    def _():
        out_ref[...] += part

    @pl.when(i == GRID - 1)
    def _():
        out_ref[...] = jax.nn.sigmoid(out_ref[...] + bias_ref[...])


def _pack_call(states_p):
    return pl.pallas_call(
        _pack_body,
        grid=(NPAD // 2048,),
        in_specs=[pl.BlockSpec((BATCH, 2048), lambda i: (0, i))],
        out_specs=pl.BlockSpec((1, 16, 128), lambda i: (i, 0, 0)),
        out_shape=jax.ShapeDtypeStruct((NPAD // 2048, 16, 128), jnp.int32),
    )(states_p)


def _main_call(lut_p, g2, m2, p3, w_p, bias):
    return pl.pallas_call(
        _main_body,
        grid=(GRID,),
        in_specs=[
            pl.BlockSpec((BLK, 256), lambda i: (i, 0)),
            pl.BlockSpec((K_MAX, BLK), lambda i: (0, i)),
            pl.BlockSpec((K_MAX, BLK), lambda i: (0, i)),
            pl.BlockSpec((1, 8, 128), lambda i: (i, 0, 0)),
            pl.BlockSpec((BLK, N_OUT), lambda i: (i, 0)),
            pl.BlockSpec((1, N_OUT), lambda i: (0, 0)),
        ],
        out_specs=pl.BlockSpec((BATCH, N_OUT), lambda i: (0, 0)),
        out_shape=jax.ShapeDtypeStruct((BATCH, N_OUT), jnp.float32),
    )(lut_p, g2, m2, p3, w_p, bias)


def _gather_call(packed_flat, idx_t):
    mesh = plsc.VectorSubcoreMesh(core_axis_name="c", subcore_axis_name="s")
    f = functools.partial(
        pl.kernel,
        mesh=mesh,
        out_type=jax.ShapeDtypeStruct((E,), jnp.int32),
        scratch_types=[
            pltpu.VMEM((NPAD,), jnp.int32),
            pltpu.VMEM((CHUNK,), jnp.int32),
            pltpu.VMEM((CHUNK,), jnp.int32),
        ],
        compiler_params=pltpu.CompilerParams(needs_layout_passes=False),
    )(_gather_body)
    return f(packed_flat, idx_t)


def kernel(states, adj_list, adj_list_mask, lut, W, b):
    nres0 = N_NODES - N_INPUT
    states_p = jnp.pad(states, ((0, 0), (0, NPAD - N_NODES)))
    adj_p = jnp.pad(adj_list[N_INPUT:], ((0, NRES - nres0), (0, 0)))
    idx_t = adj_p.T.reshape(-1)
    m2 = jnp.pad(adj_list_mask[N_INPUT:], ((0, NRES - nres0), (0, 0))).T
    lut_p = jnp.pad(lut[N_INPUT:], ((0, NRES - nres0), (0, 0)))
    w_p = jnp.pad(W, ((0, NRES - nres0), (0, 0)))
    bias = b.reshape(1, N_OUT)

    packed = _pack_call(states_p).reshape(-1)
    g2 = _gather_call(packed, idx_t).reshape(K_MAX, NRES)
    p3 = packed[N_INPUT:].reshape(GRID, 8, 128)
    return _main_call(lut_p, g2, m2, p3, w_p, bias)


# speedup vs baseline: 11.1282x; 1.0054x over previous
import functools

import jax
import jax.numpy as jnp
from jax import lax
from jax.experimental import pallas as pl
from jax.experimental.pallas import tpu as pltpu
from jax.experimental.pallas import tpu_sc as plsc

N_NODES = 100000
N_INPUT = 1024
K_MAX = 8
N_OUT = 128
BATCH = 32

NPAD = 100352
NRES = NPAD - N_INPUT
BLK = 1024
GRID = NRES // BLK
E = K_MAX * NRES
NW = 32
PER_W = E // NW
CHUNK = PER_W // 4
VECS = CHUNK // 16


def _pack_body(s_ref, o_ref):
    blk = s_ref[...]
    sh = lax.broadcasted_iota(jnp.int32, (BATCH, 1), 0)
    w = jnp.sum(jnp.left_shift(blk, sh), axis=0)
    o_ref[...] = w.reshape(1, 16, 128)


def _gather_body(table_hbm, idx_hbm, out_hbm, table_v, idx_v, out_v):
    wid = lax.axis_index("s") * 2 + lax.axis_index("c")
    pltpu.sync_copy(table_hbm, table_v)
    for c in range(4):
        base = wid * PER_W + c * CHUNK
        pltpu.sync_copy(idx_hbm.at[pl.ds(base, CHUNK)], idx_v)

        def body(i, carry):
            iv = idx_v[pl.ds(i * 16, 16)]
            out_v[pl.ds(i * 16, 16)] = plsc.load_gather(table_v, [iv])
            return carry

        lax.fori_loop(0, VECS, body, 0)
        pltpu.sync_copy(out_v, out_hbm.at[pl.ds(base, CHUNK)])


def _main_body(lut_ref, g_ref, m_ref, p_ref, w_ref, bias_ref, out_ref):
    i = pl.program_id(0)
    lutblk = lut_ref[...]
    s = lutblk.astype(jnp.float32)
    jidx = lax.broadcasted_iota(jnp.int32, (16, 256), 0)
    cidx = lax.broadcasted_iota(jnp.int32, (16, 256), 1)
    m16 = jnp.where(cidx // 16 == jidx,
                    jnp.left_shift(1, cidx % 16), 0).astype(jnp.float32)
    h = lax.dot_general(m16, s, (((1,), (1,)), ((), ())),
                        preferred_element_type=jnp.float32)
    h = h.astype(jnp.int32).reshape(16, 8, 128)
    words = [h[2 * j] + jnp.left_shift(h[2 * j + 1], 16) for j in range(8)]

    g = g_ref[...].reshape(K_MAX, 8, 128)
    m = m_ref[...].reshape(K_MAX, 8, 128)
    gm = [g[k] * m[k] for k in range(K_MAX)]
    msum = m[0]
    for k in range(1, K_MAX):
        msum = msum + m[k]
    nn = msum == 0
    pk = p_ref[...].reshape(8, 128)

    rows = [None] * BATCH
    for q in range(8):
        acc = None
        for k in range(K_MAX):
            quad = jnp.right_shift(gm[k], q) & 0x01010101
            acc = quad if acc is None else acc + acc + quad
        pquad = jnp.right_shift(pk, q) & 0x01010101
        for j in range(4):
            bb = q + 8 * j
            idx = jnp.right_shift(acc, 8 * j) & 255
            b5 = jnp.right_shift(idx, 5) & 1
            b6 = jnp.right_shift(idx, 6) & 1
            b7 = jnp.right_shift(idx, 7) & 1
            s01 = jnp.where(b5 == 0, words[0], words[1])
            s23 = jnp.where(b5 == 0, words[2], words[3])
            s45 = jnp.where(b5 == 0, words[4], words[5])
            s67 = jnp.where(b5 == 0, words[6], words[7])
            s03 = jnp.where(b6 == 0, s01, s23)
            s47 = jnp.where(b6 == 0, s45, s67)
            sel = jnp.where(b7 == 0, s03, s47)
            lbit = jnp.right_shift(sel, idx & 31) & 1
            old = jnp.right_shift(pquad, 8 * j) & 1
            rows[bb] = jnp.where(nn, old, lbit)
    res = jnp.stack(rows).reshape(BATCH, BLK).astype(jnp.float32)
    part = jnp.dot(res, w_ref[...], preferred_element_type=jnp.float32)

    @pl.when(i == 0)
    def _():
        out_ref[...] = part

    @pl.when(i > 0)
    def _():
        out_ref[...] += part

    @pl.when(i == GRID - 1)
    def _():
        out_ref[...] = jax.nn.sigmoid(out_ref[...] + bias_ref[...])


def _pack_call(states_p):
    return pl.pallas_call(
        _pack_body,
        grid=(NPAD // 2048,),
        in_specs=[pl.BlockSpec((BATCH, 2048), lambda i: (0, i))],
        out_specs=pl.BlockSpec((1, 16, 128), lambda i: (i, 0, 0)),
        out_shape=jax.ShapeDtypeStruct((NPAD // 2048, 16, 128), jnp.int32),
    )(states_p)


def _main_call(lut_p, g2, m2, p3, w_p, bias):
    return pl.pallas_call(
        _main_body,
        grid=(GRID,),
        in_specs=[
            pl.BlockSpec((BLK, 256), lambda i: (i, 0)),
            pl.BlockSpec((K_MAX, BLK), lambda i: (0, i)),
            pl.BlockSpec((K_MAX, BLK), lambda i: (0, i)),
            pl.BlockSpec((1, 8, 128), lambda i: (i, 0, 0)),
            pl.BlockSpec((BLK, N_OUT), lambda i: (i, 0)),
            pl.BlockSpec((1, N_OUT), lambda i: (0, 0)),
        ],
        out_specs=pl.BlockSpec((BATCH, N_OUT), lambda i: (0, 0)),
        out_shape=jax.ShapeDtypeStruct((BATCH, N_OUT), jnp.float32),
    )(lut_p, g2, m2, p3, w_p, bias)


def _gather_call(packed_flat, idx_t):
    mesh = plsc.VectorSubcoreMesh(core_axis_name="c", subcore_axis_name="s")
    f = functools.partial(
        pl.kernel,
        mesh=mesh,
        out_type=jax.ShapeDtypeStruct((E,), jnp.int32),
        scratch_types=[
            pltpu.VMEM((NPAD,), jnp.int32),
            pltpu.VMEM((CHUNK,), jnp.int32),
            pltpu.VMEM((CHUNK,), jnp.int32),
        ],
        compiler_params=pltpu.CompilerParams(needs_layout_passes=False),
    )(_gather_body)
    return f(packed_flat, idx_t)


def kernel(states, adj_list, adj_list_mask, lut, W, b):
    nres0 = N_NODES - N_INPUT
    states_p = jnp.pad(states, ((0, 0), (0, NPAD - N_NODES)))
    adj_p = jnp.pad(adj_list[N_INPUT:], ((0, NRES - nres0), (0, 0)))
    idx_t = adj_p.T.reshape(-1)
    m2 = jnp.pad(adj_list_mask[N_INPUT:], ((0, NRES - nres0), (0, 0))).T
    lut_p = jnp.pad(lut[N_INPUT:], ((0, NRES - nres0), (0, 0)))
    w_p = jnp.pad(W, ((0, NRES - nres0), (0, 0)))
    bias = b.reshape(1, N_OUT)

    packed = _pack_call(states_p).reshape(-1)
    g2 = _gather_call(packed, idx_t).reshape(K_MAX, NRES)
    p3 = packed[N_INPUT:].reshape(GRID, 8, 128)
    return _main_call(lut_p, g2, m2, p3, w_p, bias)


# speedup vs baseline: 20.4345x; 1.8363x over previous
import functools

import jax
import jax.numpy as jnp
from jax import lax
from jax.experimental import pallas as pl
from jax.experimental.pallas import tpu as pltpu
from jax.experimental.pallas import tpu_sc as plsc

N_NODES = 100000
N_INPUT = 1024
K_MAX = 8
N_OUT = 128
BATCH = 32

NPAD = 100352
NRES = NPAD - N_INPUT
BLK = 1024
GRID = NRES // BLK
E = K_MAX * NRES
NW = 32
PER_W = E // NW
CHUNK = PER_W // 4
VECS = CHUNK // 16


def _pack_body(s_ref, o_ref):
    blk = s_ref[...]
    sh = lax.broadcasted_iota(jnp.int32, (BATCH, 1), 0)
    w = jnp.sum(jnp.left_shift(blk, sh), axis=0)
    o_ref[...] = w.reshape(1, 16, 128)


def _gather_body(table_hbm, idx_hbm, out_hbm, table_v, idx_v, out_v):
    wid = lax.axis_index("s") * 2 + lax.axis_index("c")
    pltpu.sync_copy(table_hbm, table_v)
    for c in range(4):
        base = wid * PER_W + c * CHUNK
        pltpu.sync_copy(idx_hbm.at[pl.ds(base, CHUNK)], idx_v)

        def body(i, carry):
            iv = idx_v[pl.ds(i * 16, 16)]
            out_v[pl.ds(i * 16, 16)] = plsc.load_gather(table_v, [iv])
            return carry

        lax.fori_loop(0, VECS, body, 0)
        pltpu.sync_copy(out_v, out_hbm.at[pl.ds(base, CHUNK)])


def _main_body(lut_ref, g_ref, m_ref, p_ref, w_ref, bias_ref, out_ref):
    i = pl.program_id(0)
    rowid = i * BLK + lax.broadcasted_iota(jnp.int32, (BLK, 1), 0)
    wblk = jnp.where(rowid < N_NODES - N_INPUT, w_ref[...], 0.0)
    lutblk = lut_ref[...]
    s = lutblk.astype(jnp.float32)
    jidx = lax.broadcasted_iota(jnp.int32, (16, 256), 0)
    cidx = lax.broadcasted_iota(jnp.int32, (16, 256), 1)
    m16 = jnp.where(cidx // 16 == jidx,
                    jnp.left_shift(1, cidx % 16), 0).astype(jnp.float32)
    h = lax.dot_general(m16, s, (((1,), (1,)), ((), ())),
                        preferred_element_type=jnp.float32)
    h = h.astype(jnp.int32).reshape(16, 8, 128)
    words = [h[2 * j] + jnp.left_shift(h[2 * j + 1], 16) for j in range(8)]

    g = g_ref[...].reshape(K_MAX, 8, 128)
    m = m_ref[...].reshape(K_MAX, 8, 128)
    gm = [g[k] * m[k] for k in range(K_MAX)]
    msum = m[0]
    for k in range(1, K_MAX):
        msum = msum + m[k]
    nn = msum == 0
    pk = p_ref[...].reshape(8, 128)

    rows = [None] * BATCH
    for q in range(8):
        acc = None
        for k in range(K_MAX):
            quad = jnp.right_shift(gm[k], q) & 0x01010101
            acc = quad if acc is None else acc + acc + quad
        pquad = jnp.right_shift(pk, q) & 0x01010101
        for j in range(4):
            bb = q + 8 * j
            idx = jnp.right_shift(acc, 8 * j) & 255
            b5 = jnp.right_shift(idx, 5) & 1
            b6 = jnp.right_shift(idx, 6) & 1
            b7 = jnp.right_shift(idx, 7) & 1
            s01 = jnp.where(b5 == 0, words[0], words[1])
            s23 = jnp.where(b5 == 0, words[2], words[3])
            s45 = jnp.where(b5 == 0, words[4], words[5])
            s67 = jnp.where(b5 == 0, words[6], words[7])
            s03 = jnp.where(b6 == 0, s01, s23)
            s47 = jnp.where(b6 == 0, s45, s67)
            sel = jnp.where(b7 == 0, s03, s47)
            lbit = jnp.right_shift(sel, idx & 31) & 1
            old = jnp.right_shift(pquad, 8 * j) & 1
            rows[bb] = jnp.where(nn, old, lbit)
    res = jnp.stack(rows).reshape(BATCH, BLK).astype(jnp.float32)
    part = jnp.dot(res, wblk, preferred_element_type=jnp.float32)

    @pl.when(i == 0)
    def _():
        out_ref[...] = part

    @pl.when(i > 0)
    def _():
        out_ref[...] += part

    @pl.when(i == GRID - 1)
    def _():
        out_ref[...] = jax.nn.sigmoid(out_ref[...] + bias_ref[...])


def _pack_call(states_p):
    return pl.pallas_call(
        _pack_body,
        grid=(NPAD // 2048,),
        in_specs=[pl.BlockSpec((BATCH, 2048), lambda i: (0, i))],
        out_specs=pl.BlockSpec((1, 16, 128), lambda i: (i, 0, 0)),
        out_shape=jax.ShapeDtypeStruct((NPAD // 2048, 16, 128), jnp.int32),
    )(states_p)


def _main_call(lut_p, g2, m2, p3, w_p, bias):
    return pl.pallas_call(
        _main_body,
        grid=(GRID,),
        in_specs=[
            pl.BlockSpec((BLK, 256), lambda i: (i + 1, 0)),
            pl.BlockSpec((K_MAX, BLK), lambda i: (0, i)),
            pl.BlockSpec((K_MAX, BLK), lambda i: (0, i)),
            pl.BlockSpec((1, 8, 128), lambda i: (i, 0, 0)),
            pl.BlockSpec((BLK, N_OUT), lambda i: (i, 0)),
            pl.BlockSpec((1, N_OUT), lambda i: (0, 0)),
        ],
        out_specs=pl.BlockSpec((BATCH, N_OUT), lambda i: (0, 0)),
        out_shape=jax.ShapeDtypeStruct((BATCH, N_OUT), jnp.float32),
    )(lut_p, g2, m2, p3, w_p, bias)


def _gather_call(packed_flat, idx_t):
    mesh = plsc.VectorSubcoreMesh(core_axis_name="c", subcore_axis_name="s")
    f = functools.partial(
        pl.kernel,
        mesh=mesh,
        out_type=jax.ShapeDtypeStruct((E,), jnp.int32),
        scratch_types=[
            pltpu.VMEM((NPAD,), jnp.int32),
            pltpu.VMEM((CHUNK,), jnp.int32),
            pltpu.VMEM((CHUNK,), jnp.int32),
        ],
        compiler_params=pltpu.CompilerParams(needs_layout_passes=False),
    )(_gather_body)
    return f(packed_flat, idx_t)


def kernel(states, adj_list, adj_list_mask, lut, W, b):
    nres0 = N_NODES - N_INPUT
    adj_p = jnp.pad(adj_list[N_INPUT:], ((0, NRES - nres0), (0, 0)))
    idx_t = adj_p.T.reshape(-1)
    m2 = jnp.pad(adj_list_mask[N_INPUT:], ((0, NRES - nres0), (0, 0))).T
    bias = b.reshape(1, N_OUT)

    packed = _pack_call(states).reshape(-1)
    g2 = _gather_call(packed, idx_t).reshape(K_MAX, NRES)
    p3 = packed[N_INPUT:].reshape(GRID, 8, 128)
    return _main_call(lut, g2, m2, p3, W, bias)
